# baseline (speedup 1.0000x reference)
import functools

import jax
import jax.numpy as jnp
from jax import lax
from jax.experimental import pallas as pl
from jax.experimental.pallas import tpu as pltpu
from jax.experimental.pallas import tpu_sc as plsc

BM = 256


def _route_body(x_ref, gw_ref, gb_ref, dest_ref, be_ref,
                acc_ref, eidx_s, grank_s, gs_s, *, E, nblocks, nblk_pad):
    p = pl.program_id(0)
    i = pl.program_id(1)

    @pl.when(jnp.logical_and(p == 0, i == 0))
    def _():
        acc_ref[...] = jnp.zeros_like(acc_ref)

    @pl.when(p == 0)
    def _pass0():
        x = x_ref[...]
        glog = jnp.dot(x, gw_ref[...], preferred_element_type=jnp.float32)
        glog = glog + gb_ref[...]
        eidx = jnp.argmax(glog, axis=-1).astype(jnp.int32)
        e_iota = lax.broadcasted_iota(jnp.int32, (1, E), 1)
        onehot = (eidx[:, None] == e_iota).astype(jnp.float32)
        r = lax.broadcasted_iota(jnp.int32, (BM, BM), 0)
        c = lax.broadcasted_iota(jnp.int32, (BM, BM), 1)
        tri = (c < r).astype(jnp.float32)
        ranks_local = jnp.dot(tri, onehot, preferred_element_type=jnp.float32)
        counts_prev = acc_ref[...]
        grank = jnp.sum(onehot * (ranks_local + counts_prev), axis=-1)
        acc_ref[...] = counts_prev + jnp.sum(onehot, axis=0, keepdims=True)
        eidx_s[i] = eidx.reshape(1, BM)
        grank_s[i] = grank.astype(jnp.int32).reshape(1, BM)

        @pl.when(i == nblocks - 1)
        def _finalize():
            counts = acc_ref[...]
            pad_blocks = jnp.ceil(counts / BM)
            re_ = lax.broadcasted_iota(jnp.int32, (E, E), 0)
            ce_ = lax.broadcasted_iota(jnp.int32, (E, E), 1)
            tinc = (re_ <= ce_).astype(jnp.float32)
            ends = jnp.dot(pad_blocks, tinc,
                           preferred_element_type=jnp.float32)
            starts_rows = (ends - pad_blocks) * BM
            gs_s[...] = starts_rows.astype(jnp.int32)
            j_iota = lax.broadcasted_iota(jnp.int32, (nblk_pad, 1), 0)
            be = jnp.sum((j_iota >= ends.astype(jnp.int32)).astype(jnp.int32),
                         axis=-1)
            be_ref[...] = jnp.clip(be, 0, E - 1).reshape(1, nblk_pad)

    @pl.when(p == 1)
    def _pass1():
        e = eidx_s[i]
        g = grank_s[i]
        acc = jnp.zeros_like(g)
        for ee in range(E):
            acc = jnp.where(e == ee, gs_s[0, ee], acc)
        dest_ref[...] = (g + acc).reshape(1, 1, BM)


def _route(inputs, gate_W, gate_b2, *, N, D, E, nblk_pad):
    nblocks = N // BM
    return pl.pallas_call(
        functools.partial(_route_body, E=E, nblocks=nblocks, nblk_pad=nblk_pad),
        grid=(2, nblocks),
        in_specs=[
            pl.BlockSpec((BM, D), lambda p, i: ((1 - p) * i, 0)),
            pl.BlockSpec((D, E), lambda p, i: (0, 0)),
            pl.BlockSpec((1, E), lambda p, i: (0, 0)),
        ],
        out_specs=[
            pl.BlockSpec((1, 1, BM), lambda p, i: (i, 0, 0)),
            pl.BlockSpec((1, nblk_pad), lambda p, i: (0, 0)),
        ],
        out_shape=[
            jax.ShapeDtypeStruct((nblocks, 1, BM), jnp.int32),
            jax.ShapeDtypeStruct((1, nblk_pad), jnp.int32),
        ],
        scratch_shapes=[
            pltpu.VMEM((1, E), jnp.float32),
            pltpu.VMEM((nblocks, 1, BM), jnp.int32),
            pltpu.VMEM((nblocks, 1, BM), jnp.int32),
            pltpu.VMEM((1, E), jnp.int32),
        ],
    )(inputs, gate_W, gate_b2)


def _make_scatter(N, D, NPAD):
    info = plsc.get_sparse_core_info()
    NC, NS = info.num_cores, info.num_subcores
    NW = NC * NS
    n_per_w = N // NW
    CH = 64
    n_ch = n_per_w // CH

    @functools.partial(
        pl.kernel,
        out_type=jax.ShapeDtypeStruct((NPAD, D), jnp.float32),
        mesh=plsc.VectorSubcoreMesh(core_axis_name="c", subcore_axis_name="s"),
        compiler_params=pltpu.CompilerParams(needs_layout_passes=False),
        scratch_types=[
            pltpu.VMEM((CH,), jnp.int32),
            pltpu.VMEM((CH,), jnp.int32),
            pltpu.VMEM((CH, D), jnp.float32),
            pltpu.VMEM((CH, D), jnp.float32),
            pltpu.SemaphoreType.DMA,
            pltpu.SemaphoreType.DMA,
            pltpu.SemaphoreType.DMA,
            pltpu.SemaphoreType.DMA,
            pltpu.SemaphoreType.DMA,
            pltpu.SemaphoreType.DMA,
        ],
    )
    def scatter_x(x_hbm, dest_hbm, xs_hbm,
                  d_v0, d_v1, r_v0, r_v1, sd0, sd1, sr0, sr1, ss0, ss1):
        wid = lax.axis_index("s") * NC + lax.axis_index("c")
        base = wid * n_per_w
        bufs = [(d_v0, r_v0, sd0, sr0, ss0), (d_v1, r_v1, sd1, sr1, ss1)]
        pending = [None, None]
        for ci in range(n_ch):
            b = ci % 2
            d_v, r_v, sd, sr, ss = bufs[b]
            if pending[b] is not None:
                pending[b].wait()
            off = base + ci * CH
            hd = pltpu.async_copy(dest_hbm.at[pl.ds(off, CH)], d_v, sd)
            hr = pltpu.async_copy(x_hbm.at[pl.ds(off, CH)], r_v, sr)
            hd.wait()
            hr.wait()
            pending[b] = pltpu.async_copy(r_v, xs_hbm.at[d_v], ss)
        for h in pending:
            if h is not None:
                h.wait()

    return scatter_x


def _make_gather(N, C, NPAD):
    info = plsc.get_sparse_core_info()
    NC, NS = info.num_cores, info.num_subcores
    NW = NC * NS
    n_per_w = N // NW
    CH = 32
    n_ch = n_per_w // CH

    @functools.partial(
        pl.kernel,
        out_type=jax.ShapeDtypeStruct((N, C), jnp.float32),
        mesh=plsc.VectorSubcoreMesh(core_axis_name="c", subcore_axis_name="s"),
        compiler_params=pltpu.CompilerParams(needs_layout_passes=False),
        scratch_types=[
            pltpu.VMEM((CH,), jnp.int32),
            pltpu.VMEM((CH,), jnp.int32),
            pltpu.VMEM((CH, C), jnp.float32),
            pltpu.VMEM((CH, C), jnp.float32),
            pltpu.SemaphoreType.DMA,
            pltpu.SemaphoreType.DMA,
            pltpu.SemaphoreType.DMA,
            pltpu.SemaphoreType.DMA,
            pltpu.SemaphoreType.DMA,
            pltpu.SemaphoreType.DMA,
        ],
    )
    def gather_y(y_hbm, dest_hbm, out_hbm,
                 d_v0, d_v1, r_v0, r_v1, sd0, sd1, sg0, sg1, st0, st1):
        wid = lax.axis_index("s") * NC + lax.axis_index("c")
        base = wid * n_per_w
        bufs = [(d_v0, r_v0, sd0, sg0, st0), (d_v1, r_v1, sd1, sg1, st1)]
        pending = [None, None]
        for ci in range(n_ch):
            b = ci % 2
            d_v, r_v, sd, sg, st = bufs[b]
            if pending[b] is not None:
                pending[b].wait()
            off = base + ci * CH
            hd = pltpu.async_copy(dest_hbm.at[pl.ds(off, CH)], d_v, sd)
            hd.wait()
            hg = pltpu.async_copy(y_hbm.at[d_v], r_v, sg)
            hg.wait()
            pending[b] = pltpu.async_copy(r_v, out_hbm.at[pl.ds(off, CH)], st)
        for h in pending:
            if h is not None:
                h.wait()

    return gather_y


def _mm_body(be_ref, xs_ref, w_ref, b_ref, o_ref):
    y = jnp.dot(xs_ref[...], w_ref[0], preferred_element_type=jnp.float32)
    o_ref[...] = jax.nn.softmax(y + b_ref[0], axis=-1)


def _grouped_mm(be_arr, x_sorted, expert_W, expert_b, *, D, C, NPAD):
    nblk = NPAD // BM
    grid_spec = pltpu.PrefetchScalarGridSpec(
        num_scalar_prefetch=1,
        grid=(nblk,),
        in_specs=[
            pl.BlockSpec((BM, D), lambda j, be: (j, 0)),
            pl.BlockSpec((1, D, C), lambda j, be: (be[j], 0, 0)),
            pl.BlockSpec((1, 1, C), lambda j, be: (be[j], 0, 0)),
        ],
        out_specs=pl.BlockSpec((BM, C), lambda j, be: (j, 0)),
    )
    return pl.pallas_call(
        _mm_body,
        grid_spec=grid_spec,
        out_shape=jax.ShapeDtypeStruct((NPAD, C), jnp.float32),
    )(be_arr, x_sorted, expert_W, expert_b.reshape(expert_b.shape[0], 1, C))


def kernel(inputs, expert_W, expert_b, gate_W, gate_b):
    N, D = inputs.shape
    E, _, C = expert_W.shape
    NPAD = N + E * BM
    nblk_pad = 128

    dest3, be2 = _route(
        inputs, gate_W, gate_b.reshape(1, E), N=N, D=D, E=E, nblk_pad=nblk_pad)
    dest = dest3.reshape(N)
    be_arr = be2.reshape(nblk_pad)

    x_sorted = _make_scatter(N, D, NPAD)(inputs, dest)
    y_sorted = _grouped_mm(be_arr, x_sorted, expert_W, expert_b,
                           D=D, C=C, NPAD=NPAD)
    out = _make_gather(N, C, NPAD)(y_sorted, dest)
    return out


# speedup vs baseline: 1.0471x; 1.0471x over previous
import functools

import jax
import jax.numpy as jnp
from jax import lax
from jax.experimental import pallas as pl
from jax.experimental.pallas import tpu as pltpu
from jax.experimental.pallas import tpu_sc as plsc

BM = 256


def _route_body(x_ref, gw_ref, gb_ref, tri_ref, eidx_ref, grank_ref, gs_ref,
                be_ref, acc_ref, *, E, nblocks, nblk_pad):
    i = pl.program_id(0)

    @pl.when(i == 0)
    def _():
        acc_ref[...] = jnp.zeros_like(acc_ref)

    x = x_ref[...]
    glog = jnp.dot(x, gw_ref[...], preferred_element_type=jnp.float32)
    glog = glog + gb_ref[...]
    eidx = jnp.argmax(glog, axis=-1).astype(jnp.int32)
    e_iota = lax.broadcasted_iota(jnp.int32, (1, E), 1)
    onehot = (eidx[:, None] == e_iota).astype(jnp.float32)
    ranks_local = jnp.dot(tri_ref[...], onehot,
                          preferred_element_type=jnp.float32)
    counts_prev = acc_ref[...]
    grank = jnp.sum(onehot * (ranks_local + counts_prev), axis=-1)
    acc_ref[...] = counts_prev + jnp.sum(onehot, axis=0, keepdims=True)
    eidx_ref[...] = eidx.reshape(1, 1, BM)
    grank_ref[...] = grank.astype(jnp.int32).reshape(1, 1, BM)

    @pl.when(i == nblocks - 1)
    def _finalize():
        counts = acc_ref[...]
        pad_blocks = jnp.ceil(counts / BM)
        re_ = lax.broadcasted_iota(jnp.int32, (E, E), 0)
        ce_ = lax.broadcasted_iota(jnp.int32, (E, E), 1)
        tinc = (re_ <= ce_).astype(jnp.float32)
        ends = jnp.dot(pad_blocks, tinc,
                       preferred_element_type=jnp.float32)
        starts_rows = (ends - pad_blocks) * BM
        gs_ref[...] = jnp.concatenate(
            [starts_rows, jnp.zeros_like(starts_rows)],
            axis=-1).astype(jnp.int32)
        ends_i = ends.astype(jnp.int32)
        j_iota = lax.broadcasted_iota(jnp.int32, (nblk_pad, 1), 0)
        be = jnp.clip(jnp.sum((j_iota >= ends_i).astype(jnp.int32), axis=-1),
                      0, E - 1)
        used = ends_i[0, E - 1]
        jr = lax.broadcasted_iota(jnp.int32, (nblk_pad,), 0)
        be_ref[...] = jnp.where(jr == nblk_pad - 1, used,
                                be).reshape(1, nblk_pad)


def _route(inputs, gate_W, gate_b2, tri, *, N, D, E, nblk_pad):
    nblocks = N // BM
    return pl.pallas_call(
        functools.partial(_route_body, E=E, nblocks=nblocks, nblk_pad=nblk_pad),
        grid=(nblocks,),
        in_specs=[
            pl.BlockSpec((BM, D), lambda i: (i, 0)),
            pl.BlockSpec((D, E), lambda i: (0, 0)),
            pl.BlockSpec((1, E), lambda i: (0, 0)),
            pl.BlockSpec((BM, BM), lambda i: (0, 0)),
        ],
        out_specs=[
            pl.BlockSpec((1, 1, BM), lambda i: (i, 0, 0)),
            pl.BlockSpec((1, 1, BM), lambda i: (i, 0, 0)),
            pl.BlockSpec((1, 16), lambda i: (0, 0)),
            pl.BlockSpec((1, nblk_pad), lambda i: (0, 0)),
        ],
        out_shape=[
            jax.ShapeDtypeStruct((nblocks, 1, BM), jnp.int32),
            jax.ShapeDtypeStruct((nblocks, 1, BM), jnp.int32),
            jax.ShapeDtypeStruct((1, 16), jnp.int32),
            jax.ShapeDtypeStruct((1, nblk_pad), jnp.int32),
        ],
        scratch_shapes=[pltpu.VMEM((1, E), jnp.float32)],
    )(inputs, gate_W, gate_b2, tri)


def _make_scatter(N, D, NPAD):
    info = plsc.get_sparse_core_info()
    NC, NS = info.num_cores, info.num_subcores
    NW = NC * NS
    n_per_w = N // NW
    CH = 64
    n_ch = n_per_w // CH

    @functools.partial(
        pl.kernel,
        out_type=[
            jax.ShapeDtypeStruct((NPAD, D), jnp.float32),
            jax.ShapeDtypeStruct((N,), jnp.int32),
        ],
        mesh=plsc.VectorSubcoreMesh(core_axis_name="c", subcore_axis_name="s"),
        compiler_params=pltpu.CompilerParams(needs_layout_passes=False),
        scratch_types=[
            pltpu.VMEM((16,), jnp.int32),
            pltpu.VMEM((CH,), jnp.int32), pltpu.VMEM((CH,), jnp.int32),
            pltpu.VMEM((CH,), jnp.int32), pltpu.VMEM((CH,), jnp.int32),
            pltpu.VMEM((CH,), jnp.int32), pltpu.VMEM((CH,), jnp.int32),
            pltpu.VMEM((CH, D), jnp.float32), pltpu.VMEM((CH, D), jnp.float32),
            pltpu.SemaphoreType.DMA, pltpu.SemaphoreType.DMA,
            pltpu.SemaphoreType.DMA, pltpu.SemaphoreType.DMA,
            pltpu.SemaphoreType.DMA, pltpu.SemaphoreType.DMA,
            pltpu.SemaphoreType.DMA, pltpu.SemaphoreType.DMA,
            pltpu.SemaphoreType.DMA, pltpu.SemaphoreType.DMA,
            pltpu.SemaphoreType.DMA,
        ],
    )
    def scatter_x(x_hbm, eidx_hbm, grank_hbm, gs_hbm, xs_hbm, dest_hbm,
                  gs_v, e_v0, e_v1, g_v0, g_v1, d_v0, d_v1, r_v0, r_v1,
                  sgs, se0, se1, sg0, sg1, sr0, sr1, ssc0, ssc1, sst0, sst1):
        wid = lax.axis_index("s") * NC + lax.axis_index("c")
        base = wid * n_per_w
        pltpu.sync_copy(gs_hbm, gs_v)
        bufs = [(e_v0, g_v0, d_v0, r_v0, se0, sg0, sr0, ssc0, sst0),
                (e_v1, g_v1, d_v1, r_v1, se1, sg1, sr1, ssc1, sst1)]
        pend_sc = [None, None]
        pend_st = [None, None]
        for ci in range(n_ch):
            b = ci % 2
            e_v, g_v, d_v, r_v, se, sg, sr, ssc, sst = bufs[b]
            if pend_sc[b] is not None:
                pend_sc[b].wait()
                pend_st[b].wait()
            off = base + ci * CH
            he = pltpu.async_copy(eidx_hbm.at[pl.ds(off, CH)], e_v, se)
            hg = pltpu.async_copy(grank_hbm.at[pl.ds(off, CH)], g_v, sg)
            hr = pltpu.async_copy(x_hbm.at[pl.ds(off, CH)], r_v, sr)
            he.wait()
            hg.wait()
            for k in range(CH // 16):
                e16 = e_v[pl.ds(k * 16, 16)]
                g16 = g_v[pl.ds(k * 16, 16)]
                d_v[pl.ds(k * 16, 16)] = plsc.load_gather(gs_v, [e16]) + g16
            hr.wait()
            pend_sc[b] = pltpu.async_copy(r_v, xs_hbm.at[d_v], ssc)
            pend_st[b] = pltpu.async_copy(d_v, dest_hbm.at[pl.ds(off, CH)], sst)
        for h in pend_sc + pend_st:
            if h is not None:
                h.wait()

    return scatter_x


def _make_gather(N, C, NPAD):
    info = plsc.get_sparse_core_info()
    NC, NS = info.num_cores, info.num_subcores
    NW = NC * NS
    n_per_w = N // NW
    CH = 32
    n_ch = n_per_w // CH

    @functools.partial(
        pl.kernel,
        out_type=jax.ShapeDtypeStruct((N, C), jnp.float32),
        mesh=plsc.VectorSubcoreMesh(core_axis_name="c", subcore_axis_name="s"),
        compiler_params=pltpu.CompilerParams(needs_layout_passes=False),
        scratch_types=[
            pltpu.VMEM((CH,), jnp.int32), pltpu.VMEM((CH,), jnp.int32),
            pltpu.VMEM((CH,), jnp.int32),
            pltpu.VMEM((CH, C), jnp.float32), pltpu.VMEM((CH, C), jnp.float32),
            pltpu.VMEM((CH, C), jnp.float32),
            pltpu.SemaphoreType.DMA, pltpu.SemaphoreType.DMA,
            pltpu.SemaphoreType.DMA, pltpu.SemaphoreType.DMA,
            pltpu.SemaphoreType.DMA, pltpu.SemaphoreType.DMA,
            pltpu.SemaphoreType.DMA, pltpu.SemaphoreType.DMA,
            pltpu.SemaphoreType.DMA,
        ],
    )
    def gather_y(y_hbm, dest_hbm, out_hbm,
                 d_v0, d_v1, d_v2, r_v0, r_v1, r_v2,
                 sd0, sd1, sd2, sg0, sg1, sg2, st0, st1, st2):
        wid = lax.axis_index("s") * NC + lax.axis_index("c")
        base = wid * n_per_w
        bufs = [(d_v0, r_v0, sd0, sg0, st0),
                (d_v1, r_v1, sd1, sg1, st1),
                (d_v2, r_v2, sd2, sg2, st2)]
        pend_st = [None, None, None]
        for ci in range(n_ch):
            b = ci % 3
            d_v, r_v, sd, sg, st = bufs[b]
            if pend_st[b] is not None:
                pend_st[b].wait()
            off = base + ci * CH
            hd = pltpu.async_copy(dest_hbm.at[pl.ds(off, CH)], d_v, sd)
            hd.wait()
            hg = pltpu.async_copy(y_hbm.at[d_v], r_v, sg)
            hg.wait()
            pend_st[b] = pltpu.async_copy(r_v, out_hbm.at[pl.ds(off, CH)], st)
        for h in pend_st:
            if h is not None:
                h.wait()

    return gather_y


def _mm_body(be_ref, xs_ref, w_ref, b_ref, ones_ref, o_ref, *, nblk_pad):
    j = pl.program_id(0)
    used = be_ref[nblk_pad - 1]

    @pl.when(j < used)
    def _():
        y = jnp.dot(xs_ref[...], w_ref[0], preferred_element_type=jnp.float32)
        ey = jnp.exp(jnp.minimum(y + b_ref[0], 80.0))
        s = jnp.dot(ey, ones_ref[...], preferred_element_type=jnp.float32)
        o_ref[...] = ey * (1.0 / s[:, :1])


def _grouped_mm(be_arr, x_sorted, expert_W, expert_b, ones_c, *, D, C, NPAD,
                nblk_pad):
    nblk = NPAD // BM
    grid_spec = pltpu.PrefetchScalarGridSpec(
        num_scalar_prefetch=1,
        grid=(nblk,),
        in_specs=[
            pl.BlockSpec((BM, D), lambda j, be: (j, 0)),
            pl.BlockSpec((1, D, C), lambda j, be: (be[j], 0, 0)),
            pl.BlockSpec((1, 1, C), lambda j, be: (be[j], 0, 0)),
            pl.BlockSpec((C, 128), lambda j, be: (0, 0)),
        ],
        out_specs=pl.BlockSpec((BM, C), lambda j, be: (j, 0)),
    )
    return pl.pallas_call(
        functools.partial(_mm_body, nblk_pad=nblk_pad),
        grid_spec=grid_spec,
        out_shape=jax.ShapeDtypeStruct((NPAD, C), jnp.float32),
    )(be_arr, x_sorted, expert_W,
      expert_b.reshape(expert_b.shape[0], 1, C), ones_c)


def kernel(inputs, expert_W, expert_b, gate_W, gate_b):
    N, D = inputs.shape
    E, _, C = expert_W.shape
    NPAD = N + E * BM
    nblk_pad = 128

    row_i = jnp.arange(BM, dtype=jnp.int32)
    tri = (row_i[None, :] < row_i[:, None]).astype(jnp.float32)
    ones_c = jnp.ones((C, 128), jnp.float32)

    eidx3, grank3, gs2, be2 = _route(
        inputs, gate_W, gate_b.reshape(1, E), tri, N=N, D=D, E=E,
        nblk_pad=nblk_pad)
    eidx = eidx3.reshape(N)
    grank = grank3.reshape(N)
    gs = gs2.reshape(16)
    be_arr = be2.reshape(nblk_pad)

    x_sorted, dest = _make_scatter(N, D, NPAD)(inputs, eidx, grank, gs)
    y_sorted = _grouped_mm(be_arr, x_sorted, expert_W, expert_b, ones_c,
                           D=D, C=C, NPAD=NPAD, nblk_pad=nblk_pad)
    out = _make_gather(N, C, NPAD)(y_sorted, dest)
    return out


# speedup vs baseline: 1.1492x; 1.0976x over previous
import functools

import jax
import jax.numpy as jnp
from jax import lax
from jax.experimental import pallas as pl
from jax.experimental.pallas import tpu as pltpu
from jax.experimental.pallas import tpu_sc as plsc

BM = 512
RANK_BITS = 13


def _route_body(x_ref, gw_ref, gb_ref, tri_ref, pack_ref, gs_ref,
                be_ref, acc_ref, *, E, nblocks, nblk_pad):
    i = pl.program_id(0)

    @pl.when(i == 0)
    def _():
        acc_ref[...] = jnp.zeros_like(acc_ref)

    x = x_ref[...]
    glog = jnp.dot(x, gw_ref[...], preferred_element_type=jnp.float32)
    glog = glog + gb_ref[...]
    m = jnp.max(glog, axis=-1, keepdims=True)
    hit = (glog == m).astype(jnp.float32)
    ue_r = lax.broadcasted_iota(jnp.int32, (E, E), 0)
    ue_c = lax.broadcasted_iota(jnp.int32, (E, E), 1)
    triu = (ue_r < ue_c).astype(jnp.float32)
    left = jnp.dot(hit, triu, preferred_element_type=jnp.float32)
    onehot = hit * (left == 0.0).astype(jnp.float32)
    ranks_local = jnp.dot(tri_ref[...], onehot,
                          preferred_element_type=jnp.float32)
    counts_prev = acc_ref[...]
    grank = jnp.sum(onehot * (ranks_local + counts_prev), axis=-1)
    acc_ref[...] = counts_prev + jnp.sum(onehot, axis=0, keepdims=True)
    e_iota = lax.broadcasted_iota(jnp.int32, (1, E), 1).astype(jnp.float32)
    eidx = jnp.sum(onehot * e_iota, axis=-1)
    packed = eidx.astype(jnp.int32) * (1 << RANK_BITS) + grank.astype(jnp.int32)
    pack_ref[...] = packed.reshape(1, 1, BM)

    @pl.when(i == nblocks - 1)
    def _finalize():
        counts = acc_ref[...]
        pad_blocks = jnp.ceil(counts / BM)
        tinc = (ue_r <= ue_c).astype(jnp.float32)
        ends = jnp.dot(pad_blocks, tinc,
                       preferred_element_type=jnp.float32)
        starts_rows = (ends - pad_blocks) * BM
        gs_ref[...] = jnp.concatenate(
            [starts_rows, jnp.zeros_like(starts_rows)],
            axis=-1).astype(jnp.int32)
        ends_i = ends.astype(jnp.int32)
        j_iota = lax.broadcasted_iota(jnp.int32, (nblk_pad, 1), 0)
        be = jnp.clip(jnp.sum((j_iota >= ends_i).astype(jnp.int32), axis=-1),
                      0, E - 1)
        used = ends_i[0, E - 1]
        jr = lax.broadcasted_iota(jnp.int32, (nblk_pad,), 0)
        be_ref[...] = jnp.where(jr == nblk_pad - 1, used,
                                be).reshape(1, nblk_pad)


def _route(inputs, gate_W, gate_b2, tri, *, N, D, E, nblk_pad):
    nblocks = N // BM
    return pl.pallas_call(
        functools.partial(_route_body, E=E, nblocks=nblocks, nblk_pad=nblk_pad),
        grid=(nblocks,),
        in_specs=[
            pl.BlockSpec((BM, D), lambda i: (i, 0)),
            pl.BlockSpec((D, E), lambda i: (0, 0)),
            pl.BlockSpec((1, E), lambda i: (0, 0)),
            pl.BlockSpec((BM, BM), lambda i: (0, 0)),
        ],
        out_specs=[
            pl.BlockSpec((1, 1, BM), lambda i: (i, 0, 0)),
            pl.BlockSpec((1, 16), lambda i: (0, 0)),
            pl.BlockSpec((1, nblk_pad), lambda i: (0, 0)),
        ],
        out_shape=[
            jax.ShapeDtypeStruct((nblocks, 1, BM), jnp.int32),
            jax.ShapeDtypeStruct((1, 16), jnp.int32),
            jax.ShapeDtypeStruct((1, nblk_pad), jnp.int32),
        ],
        scratch_shapes=[pltpu.VMEM((1, E), jnp.float32)],
    )(inputs, gate_W, gate_b2, tri)


def _make_scatter(N, D, NPAD):
    info = plsc.get_sparse_core_info()
    NC, NS = info.num_cores, info.num_subcores
    NW = NC * NS
    n_per_w = N // NW
    CH = 64
    n_ch = n_per_w // CH

    @functools.partial(
        pl.kernel,
        out_type=[
            jax.ShapeDtypeStruct((NPAD, D), jnp.float32),
            jax.ShapeDtypeStruct((N,), jnp.int32),
        ],
        mesh=plsc.VectorSubcoreMesh(core_axis_name="c", subcore_axis_name="s"),
        compiler_params=pltpu.CompilerParams(needs_layout_passes=False),
        scratch_types=[
            pltpu.VMEM((16,), jnp.int32),
            pltpu.VMEM((CH,), jnp.int32), pltpu.VMEM((CH,), jnp.int32),
            pltpu.VMEM((CH,), jnp.int32), pltpu.VMEM((CH,), jnp.int32),
            pltpu.VMEM((CH, D), jnp.float32), pltpu.VMEM((CH, D), jnp.float32),
            pltpu.SemaphoreType.DMA, pltpu.SemaphoreType.DMA,
            pltpu.SemaphoreType.DMA, pltpu.SemaphoreType.DMA,
            pltpu.SemaphoreType.DMA, pltpu.SemaphoreType.DMA,
            pltpu.SemaphoreType.DMA, pltpu.SemaphoreType.DMA,
            pltpu.SemaphoreType.DMA,
        ],
    )
    def scatter_x(x_hbm, pack_hbm, gs_hbm, xs_hbm, dest_hbm,
                  gs_v, p_v0, p_v1, d_v0, d_v1, r_v0, r_v1,
                  sgs, sp0, sp1, sr0, sr1, ssc0, ssc1, sst0, sst1):
        wid = lax.axis_index("s") * NC + lax.axis_index("c")
        base = wid * n_per_w
        pltpu.sync_copy(gs_hbm, gs_v)
        bufs = [(p_v0, d_v0, r_v0, sp0, sr0, ssc0, sst0),
                (p_v1, d_v1, r_v1, sp1, sr1, ssc1, sst1)]
        pend_sc = [None, None]
        pend_st = [None, None]
        for ci in range(n_ch):
            b = ci % 2
            p_v, d_v, r_v, sp, sr, ssc, sst = bufs[b]
            if pend_sc[b] is not None:
                pend_sc[b].wait()
                pend_st[b].wait()
            off = base + ci * CH
            hp = pltpu.async_copy(pack_hbm.at[pl.ds(off, CH)], p_v, sp)
            hr = pltpu.async_copy(x_hbm.at[pl.ds(off, CH)], r_v, sr)
            hp.wait()
            for k in range(CH // 16):
                p16 = p_v[pl.ds(k * 16, 16)]
                e16 = lax.shift_right_logical(p16, RANK_BITS)
                g16 = lax.bitwise_and(p16, (1 << RANK_BITS) - 1)
                d_v[pl.ds(k * 16, 16)] = plsc.load_gather(gs_v, [e16]) + g16
            hr.wait()
            pend_sc[b] = pltpu.async_copy(r_v, xs_hbm.at[d_v], ssc)
            pend_st[b] = pltpu.async_copy(d_v, dest_hbm.at[pl.ds(off, CH)], sst)
        for h in pend_sc + pend_st:
            if h is not None:
                h.wait()

    return scatter_x


def _make_gather(N, C, NPAD):
    info = plsc.get_sparse_core_info()
    NC, NS = info.num_cores, info.num_subcores
    NW = NC * NS
    n_per_w = N // NW
    CH = 32
    n_ch = n_per_w // CH

    @functools.partial(
        pl.kernel,
        out_type=jax.ShapeDtypeStruct((N, C), jnp.float32),
        mesh=plsc.VectorSubcoreMesh(core_axis_name="c", subcore_axis_name="s"),
        compiler_params=pltpu.CompilerParams(needs_layout_passes=False),
        scratch_types=[
            pltpu.VMEM((CH,), jnp.int32), pltpu.VMEM((CH,), jnp.int32),
            pltpu.VMEM((CH,), jnp.int32),
            pltpu.VMEM((CH, C), jnp.float32), pltpu.VMEM((CH, C), jnp.float32),
            pltpu.VMEM((CH, C), jnp.float32),
            pltpu.SemaphoreType.DMA, pltpu.SemaphoreType.DMA,
            pltpu.SemaphoreType.DMA, pltpu.SemaphoreType.DMA,
            pltpu.SemaphoreType.DMA, pltpu.SemaphoreType.DMA,
            pltpu.SemaphoreType.DMA, pltpu.SemaphoreType.DMA,
            pltpu.SemaphoreType.DMA,
        ],
    )
    def gather_y(y_hbm, dest_hbm, out_hbm,
                 d_v0, d_v1, d_v2, r_v0, r_v1, r_v2,
                 sd0, sd1, sd2, sg0, sg1, sg2, st0, st1, st2):
        wid = lax.axis_index("s") * NC + lax.axis_index("c")
        base = wid * n_per_w
        bufs = [(d_v0, r_v0, sd0, sg0, st0),
                (d_v1, r_v1, sd1, sg1, st1),
                (d_v2, r_v2, sd2, sg2, st2)]
        pend_st = [None, None, None]
        for ci in range(n_ch):
            b = ci % 3
            d_v, r_v, sd, sg, st = bufs[b]
            if pend_st[b] is not None:
                pend_st[b].wait()
            off = base + ci * CH
            hd = pltpu.async_copy(dest_hbm.at[pl.ds(off, CH)], d_v, sd)
            hd.wait()
            hg = pltpu.async_copy(y_hbm.at[d_v], r_v, sg)
            hg.wait()
            pend_st[b] = pltpu.async_copy(r_v, out_hbm.at[pl.ds(off, CH)], st)
        for h in pend_st:
            if h is not None:
                h.wait()

    return gather_y


def _mm_body(be_ref, xs_ref, w_ref, b_ref, ones_ref, o_ref, *, nblk_pad):
    j = pl.program_id(0)
    used = be_ref[nblk_pad - 1]

    @pl.when(j < used)
    def _():
        y = jnp.dot(xs_ref[...], w_ref[0], preferred_element_type=jnp.float32)
        ey = jnp.exp(jnp.minimum(y + b_ref[0], 80.0))
        s = jnp.dot(ey, ones_ref[...], preferred_element_type=jnp.float32)
        o_ref[...] = ey * (1.0 / s[:, :1])


def _grouped_mm(be_arr, x_sorted, expert_W, expert_b, ones_c, *, D, C, NPAD,
                nblk_pad):
    nblk = NPAD // BM
    grid_spec = pltpu.PrefetchScalarGridSpec(
        num_scalar_prefetch=1,
        grid=(nblk,),
        in_specs=[
            pl.BlockSpec((BM, D), lambda j, be: (j, 0)),
            pl.BlockSpec((1, D, C), lambda j, be: (be[j], 0, 0)),
            pl.BlockSpec((1, 1, C), lambda j, be: (be[j], 0, 0)),
            pl.BlockSpec((C, 128), lambda j, be: (0, 0)),
        ],
        out_specs=pl.BlockSpec((BM, C), lambda j, be: (j, 0)),
    )
    return pl.pallas_call(
        functools.partial(_mm_body, nblk_pad=nblk_pad),
        grid_spec=grid_spec,
        out_shape=jax.ShapeDtypeStruct((NPAD, C), jnp.float32),
    )(be_arr, x_sorted, expert_W,
      expert_b.reshape(expert_b.shape[0], 1, C), ones_c)


def kernel(inputs, expert_W, expert_b, gate_W, gate_b):
    N, D = inputs.shape
    E, _, C = expert_W.shape
    NPAD = N + E * BM
    nblk_pad = 128

    row_i = jnp.arange(BM, dtype=jnp.int32)
    tri = (row_i[None, :] < row_i[:, None]).astype(jnp.float32)
    ones_c = jnp.ones((C, 128), jnp.float32)

    pack3, gs2, be2 = _route(
        inputs, gate_W, gate_b.reshape(1, E), tri, N=N, D=D, E=E,
        nblk_pad=nblk_pad)
    pack = pack3.reshape(N)
    gs = gs2.reshape(16)
    be_arr = be2.reshape(nblk_pad)

    x_sorted, dest = _make_scatter(N, D, NPAD)(inputs, pack, gs)
    y_sorted = _grouped_mm(be_arr, x_sorted, expert_W, expert_b, ones_c,
                           D=D, C=C, NPAD=NPAD, nblk_pad=nblk_pad)
    out = _make_gather(N, C, NPAD)(y_sorted, dest)
    return out


# speedup vs baseline: 1.1603x; 1.0097x over previous
import functools

import jax
import jax.numpy as jnp
from jax import lax
from jax.experimental import pallas as pl
from jax.experimental.pallas import tpu as pltpu
from jax.experimental.pallas import tpu_sc as plsc

BM = 512
RANK_BITS = 13


def _route_body(x_ref, gw_ref, gb_ref, tri_ref, pack_ref, gs_ref,
                be_ref, acc_ref, *, E, nblocks, nblk_pad):
    i = pl.program_id(0)

    @pl.when(i == 0)
    def _():
        acc_ref[...] = jnp.zeros_like(acc_ref)

    x = x_ref[...]
    glog = jnp.dot(x, gw_ref[...], preferred_element_type=jnp.float32)
    glog = glog + gb_ref[...]
    m = jnp.max(glog, axis=-1, keepdims=True)
    hit = (glog == m).astype(jnp.float32)
    ue_r = lax.broadcasted_iota(jnp.int32, (E, E), 0)
    ue_c = lax.broadcasted_iota(jnp.int32, (E, E), 1)
    triu = (ue_r < ue_c).astype(jnp.float32)
    left = jnp.dot(hit, triu, preferred_element_type=jnp.float32)
    onehot = hit * (left == 0.0).astype(jnp.float32)
    ranks_local = jnp.dot(tri_ref[...], onehot,
                          preferred_element_type=jnp.float32)
    counts_prev = acc_ref[...]
    grank = jnp.sum(onehot * (ranks_local + counts_prev), axis=-1)
    acc_ref[...] = counts_prev + jnp.sum(onehot, axis=0, keepdims=True)
    e_iota = lax.broadcasted_iota(jnp.int32, (1, E), 1).astype(jnp.float32)
    eidx = jnp.sum(onehot * e_iota, axis=-1)
    packed = eidx.astype(jnp.int32) * (1 << RANK_BITS) + grank.astype(jnp.int32)
    pack_ref[...] = packed.reshape(1, 1, BM)

    @pl.when(i == nblocks - 1)
    def _finalize():
        counts = acc_ref[...]
        pad_blocks = jnp.ceil(counts / BM)
        tinc = (ue_r <= ue_c).astype(jnp.float32)
        ends = jnp.dot(pad_blocks, tinc,
                       preferred_element_type=jnp.float32)
        starts_rows = (ends - pad_blocks) * BM
        gs_ref[...] = jnp.concatenate(
            [starts_rows, jnp.zeros_like(starts_rows)],
            axis=-1).astype(jnp.int32)
        ends_i = ends.astype(jnp.int32)
        j_iota = lax.broadcasted_iota(jnp.int32, (nblk_pad, 1), 0)
        be = jnp.clip(jnp.sum((j_iota >= ends_i).astype(jnp.int32), axis=-1),
                      0, E - 1)
        used = ends_i[0, E - 1]
        jr = lax.broadcasted_iota(jnp.int32, (nblk_pad,), 0)
        be_ref[...] = jnp.where(jr == nblk_pad - 1, used,
                                be).reshape(1, nblk_pad)


def _route(inputs, gate_W, gate_b2, tri, *, N, D, E, nblk_pad):
    nblocks = N // BM
    return pl.pallas_call(
        functools.partial(_route_body, E=E, nblocks=nblocks, nblk_pad=nblk_pad),
        grid=(nblocks,),
        in_specs=[
            pl.BlockSpec((BM, D), lambda i: (i, 0)),
            pl.BlockSpec((D, E), lambda i: (0, 0)),
            pl.BlockSpec((1, E), lambda i: (0, 0)),
            pl.BlockSpec((BM, BM), lambda i: (0, 0)),
        ],
        out_specs=[
            pl.BlockSpec((1, 1, BM), lambda i: (i, 0, 0)),
            pl.BlockSpec((1, 16), lambda i: (0, 0)),
            pl.BlockSpec((1, nblk_pad), lambda i: (0, 0)),
        ],
        out_shape=[
            jax.ShapeDtypeStruct((nblocks, 1, BM), jnp.int32),
            jax.ShapeDtypeStruct((1, 16), jnp.int32),
            jax.ShapeDtypeStruct((1, nblk_pad), jnp.int32),
        ],
        scratch_shapes=[pltpu.VMEM((1, E), jnp.float32)],
    )(inputs, gate_W, gate_b2, tri)


def _make_scatter(N, D, NPAD):
    info = plsc.get_sparse_core_info()
    NC, NS = info.num_cores, info.num_subcores
    NW = NC * NS
    n_per_w = N // NW
    CH = 64
    n_ch = n_per_w // CH

    @functools.partial(
        pl.kernel,
        out_type=[
            jax.ShapeDtypeStruct((NPAD, D), jnp.float32),
            jax.ShapeDtypeStruct((N,), jnp.int32),
        ],
        mesh=plsc.VectorSubcoreMesh(core_axis_name="c", subcore_axis_name="s"),
        compiler_params=pltpu.CompilerParams(needs_layout_passes=False),
        scratch_types=[
            pltpu.VMEM((16,), jnp.int32),
            pltpu.VMEM((CH,), jnp.int32), pltpu.VMEM((CH,), jnp.int32),
            pltpu.VMEM((CH,), jnp.int32), pltpu.VMEM((CH,), jnp.int32),
            pltpu.VMEM((CH, D), jnp.float32), pltpu.VMEM((CH, D), jnp.float32),
            pltpu.SemaphoreType.DMA, pltpu.SemaphoreType.DMA,
            pltpu.SemaphoreType.DMA, pltpu.SemaphoreType.DMA,
            pltpu.SemaphoreType.DMA, pltpu.SemaphoreType.DMA,
            pltpu.SemaphoreType.DMA, pltpu.SemaphoreType.DMA,
            pltpu.SemaphoreType.DMA,
        ],
    )
    def scatter_x(x_hbm, pack_hbm, gs_hbm, xs_hbm, dest_hbm,
                  gs_v, p_v0, p_v1, d_v0, d_v1, r_v0, r_v1,
                  sgs, sp0, sp1, sr0, sr1, ssc0, ssc1, sst0, sst1):
        wid = lax.axis_index("s") * NC + lax.axis_index("c")
        base = wid * n_per_w
        pltpu.sync_copy(gs_hbm, gs_v)
        bufs = [(p_v0, d_v0, r_v0, sp0, sr0, ssc0, sst0),
                (p_v1, d_v1, r_v1, sp1, sr1, ssc1, sst1)]
        pend_sc = [None, None]
        pend_st = [None, None]
        for ci in range(n_ch):
            b = ci % 2
            p_v, d_v, r_v, sp, sr, ssc, sst = bufs[b]
            if pend_sc[b] is not None:
                pend_sc[b].wait()
                pend_st[b].wait()
            off = base + ci * CH
            hp = pltpu.async_copy(pack_hbm.at[pl.ds(off, CH)], p_v, sp)
            hr = pltpu.async_copy(x_hbm.at[pl.ds(off, CH)], r_v, sr)
            hp.wait()
            for k in range(CH // 16):
                p16 = p_v[pl.ds(k * 16, 16)]
                e16 = lax.shift_right_logical(p16, RANK_BITS)
                g16 = lax.bitwise_and(p16, (1 << RANK_BITS) - 1)
                d_v[pl.ds(k * 16, 16)] = plsc.load_gather(gs_v, [e16]) + g16
            hr.wait()
            pend_sc[b] = pltpu.async_copy(r_v, xs_hbm.at[d_v], ssc)
            pend_st[b] = pltpu.async_copy(d_v, dest_hbm.at[pl.ds(off, CH)], sst)
        for h in pend_sc + pend_st:
            if h is not None:
                h.wait()

    return scatter_x


def _make_gather(N, C, NPAD):
    info = plsc.get_sparse_core_info()
    NC, NS = info.num_cores, info.num_subcores
    NW = NC * NS
    n_per_w = N // NW
    CH = 32
    n_ch = n_per_w // CH

    @functools.partial(
        pl.kernel,
        out_type=jax.ShapeDtypeStruct((N, C), jnp.float32),
        mesh=plsc.VectorSubcoreMesh(core_axis_name="c", subcore_axis_name="s"),
        compiler_params=pltpu.CompilerParams(needs_layout_passes=False),
        scratch_types=[
            pltpu.VMEM((CH,), jnp.int32), pltpu.VMEM((CH,), jnp.int32),
            pltpu.VMEM((CH,), jnp.int32),
            pltpu.VMEM((CH, C), jnp.float32), pltpu.VMEM((CH, C), jnp.float32),
            pltpu.VMEM((CH, C), jnp.float32),
            pltpu.SemaphoreType.DMA, pltpu.SemaphoreType.DMA,
            pltpu.SemaphoreType.DMA, pltpu.SemaphoreType.DMA,
            pltpu.SemaphoreType.DMA, pltpu.SemaphoreType.DMA,
            pltpu.SemaphoreType.DMA, pltpu.SemaphoreType.DMA,
            pltpu.SemaphoreType.DMA,
        ],
    )
    def gather_y(y_hbm, dest_hbm, out_hbm,
                 d_v0, d_v1, d_v2, r_v0, r_v1, r_v2,
                 sd0, sd1, sd2, sg0, sg1, sg2, st0, st1, st2):
        wid = lax.axis_index("s") * NC + lax.axis_index("c")
        base = wid * n_per_w
        bufs = [(d_v0, r_v0, sd0, sg0, st0),
                (d_v1, r_v1, sd1, sg1, st1),
                (d_v2, r_v2, sd2, sg2, st2)]
        pend_st = [None, None, None]
        for ci in range(n_ch):
            b = ci % 3
            d_v, r_v, sd, sg, st = bufs[b]
            if pend_st[b] is not None:
                pend_st[b].wait()
            off = base + ci * CH
            hd = pltpu.async_copy(dest_hbm.at[pl.ds(off, CH)], d_v, sd)
            hd.wait()
            hg = pltpu.async_copy(y_hbm.at[d_v], r_v, sg)
            hg.wait()
            pend_st[b] = pltpu.async_copy(r_v, out_hbm.at[pl.ds(off, CH)], st)
        for h in pend_st:
            if h is not None:
                h.wait()

    return gather_y


def _mm_body(be_ref, xs_ref, w_ref, b_ref, ones_ref, o_ref, *, nblk_pad):
    j = pl.program_id(0)
    used = be_ref[nblk_pad - 1]

    @pl.when(j < used)
    def _():
        e = be_ref[j]
        y = jnp.dot(xs_ref[...], w_ref[e], preferred_element_type=jnp.float32)
        ey = jnp.exp(jnp.minimum(y + b_ref[e], 80.0))
        s = jnp.dot(ey, ones_ref[...], preferred_element_type=jnp.float32)
        o_ref[...] = ey * (1.0 / s[:, :1])


def _grouped_mm(be_arr, x_sorted, expert_W, expert_b, ones_c, *, D, C, NPAD,
                nblk_pad):
    nblk = NPAD // BM
    E = expert_W.shape[0]
    grid_spec = pltpu.PrefetchScalarGridSpec(
        num_scalar_prefetch=1,
        grid=(nblk,),
        in_specs=[
            pl.BlockSpec((BM, D), lambda j, be: (j, 0)),
            pl.BlockSpec((E, D, C), lambda j, be: (0, 0, 0)),
            pl.BlockSpec((E, 1, C), lambda j, be: (0, 0, 0)),
            pl.BlockSpec((C, 128), lambda j, be: (0, 0)),
        ],
        out_specs=pl.BlockSpec((BM, C), lambda j, be: (j, 0)),
    )
    return pl.pallas_call(
        functools.partial(_mm_body, nblk_pad=nblk_pad),
        grid_spec=grid_spec,
        out_shape=jax.ShapeDtypeStruct((NPAD, C), jnp.float32),
    )(be_arr, x_sorted, expert_W,
      expert_b.reshape(expert_b.shape[0], 1, C), ones_c)


def kernel(inputs, expert_W, expert_b, gate_W, gate_b):
    N, D = inputs.shape
    E, _, C = expert_W.shape
    NPAD = N + E * BM
    nblk_pad = 128

    row_i = jnp.arange(BM, dtype=jnp.int32)
    tri = (row_i[None, :] < row_i[:, None]).astype(jnp.float32)
    ones_c = jnp.ones((C, 128), jnp.float32)

    pack3, gs2, be2 = _route(
        inputs, gate_W, gate_b.reshape(1, E), tri, N=N, D=D, E=E,
        nblk_pad=nblk_pad)
    pack = pack3.reshape(N)
    gs = gs2.reshape(16)
    be_arr = be2.reshape(nblk_pad)

    x_sorted, dest = _make_scatter(N, D, NPAD)(inputs, pack, gs)
    y_sorted = _grouped_mm(be_arr, x_sorted, expert_W, expert_b, ones_c,
                           D=D, C=C, NPAD=NPAD, nblk_pad=nblk_pad)
    out = _make_gather(N, C, NPAD)(y_sorted, dest)
    return out


# speedup vs baseline: 1.2042x; 1.0378x over previous
import functools

import jax
import jax.numpy as jnp
from jax import lax
from jax.experimental import pallas as pl
from jax.experimental.pallas import tpu as pltpu
from jax.experimental.pallas import tpu_sc as plsc

BM = 512
RANK_BITS = 13


def _route_body(x_ref, gw_ref, gb_ref, pack_ref, gs_ref,
                be_ref, acc_ref, tri_ref, *, E, nblocks, nblk_pad):
    i = pl.program_id(0)

    @pl.when(i == 0)
    def _():
        acc_ref[...] = jnp.zeros_like(acc_ref)
        tr = lax.broadcasted_iota(jnp.int32, (BM, BM), 0)
        tc = lax.broadcasted_iota(jnp.int32, (BM, BM), 1)
        tri_ref[...] = (tc < tr).astype(jnp.float32)

    x = x_ref[...]
    glog = jnp.dot(x, gw_ref[...], preferred_element_type=jnp.float32)
    glog = glog + gb_ref[...]
    m = jnp.max(glog, axis=-1, keepdims=True)
    hit = (glog == m).astype(jnp.float32)
    ue_r = lax.broadcasted_iota(jnp.int32, (E, E), 0)
    ue_c = lax.broadcasted_iota(jnp.int32, (E, E), 1)
    triu = (ue_r < ue_c).astype(jnp.float32)
    left = jnp.dot(hit, triu, preferred_element_type=jnp.float32)
    onehot = hit * (left == 0.0).astype(jnp.float32)
    ranks_local = jnp.dot(tri_ref[...], onehot,
                          preferred_element_type=jnp.float32)
    counts_prev = acc_ref[...]
    grank = jnp.sum(onehot * (ranks_local + counts_prev), axis=-1)
    acc_ref[...] = counts_prev + jnp.sum(onehot, axis=0, keepdims=True)
    e_iota = lax.broadcasted_iota(jnp.int32, (1, E), 1).astype(jnp.float32)
    eidx = jnp.sum(onehot * e_iota, axis=-1)
    packed = eidx.astype(jnp.int32) * (1 << RANK_BITS) + grank.astype(jnp.int32)
    pack_ref[...] = packed.reshape(1, 1, BM)

    @pl.when(i == nblocks - 1)
    def _finalize():
        counts = acc_ref[...]
        pad_blocks = jnp.ceil(counts / BM)
        tinc = (ue_r <= ue_c).astype(jnp.float32)
        ends = jnp.dot(pad_blocks, tinc,
                       preferred_element_type=jnp.float32)
        starts_rows = (ends - pad_blocks) * BM
        gs_ref[...] = jnp.concatenate(
            [starts_rows, jnp.zeros_like(starts_rows)],
            axis=-1).astype(jnp.int32)
        ends_i = ends.astype(jnp.int32)
        j_iota = lax.broadcasted_iota(jnp.int32, (nblk_pad, 1), 0)
        be = jnp.clip(jnp.sum((j_iota >= ends_i).astype(jnp.int32), axis=-1),
                      0, E - 1)
        used = ends_i[0, E - 1]
        jr = lax.broadcasted_iota(jnp.int32, (nblk_pad,), 0)
        be_ref[...] = jnp.where(jr == nblk_pad - 1, used,
                                be).reshape(1, nblk_pad)


def _route(inputs, gate_W, gate_b2, *, N, D, E, nblk_pad):
    nblocks = N // BM
    return pl.pallas_call(
        functools.partial(_route_body, E=E, nblocks=nblocks, nblk_pad=nblk_pad),
        grid=(nblocks,),
        in_specs=[
            pl.BlockSpec((BM, D), lambda i: (i, 0)),
            pl.BlockSpec((D, E), lambda i: (0, 0)),
            pl.BlockSpec((1, E), lambda i: (0, 0)),
        ],
        out_specs=[
            pl.BlockSpec((1, 1, BM), lambda i: (i, 0, 0)),
            pl.BlockSpec((1, 16), lambda i: (0, 0)),
            pl.BlockSpec((1, nblk_pad), lambda i: (0, 0)),
        ],
        out_shape=[
            jax.ShapeDtypeStruct((nblocks, 1, BM), jnp.int32),
            jax.ShapeDtypeStruct((1, 16), jnp.int32),
            jax.ShapeDtypeStruct((1, nblk_pad), jnp.int32),
        ],
        scratch_shapes=[pltpu.VMEM((1, E), jnp.float32),
                        pltpu.VMEM((BM, BM), jnp.float32)],
    )(inputs, gate_W, gate_b2)


def _make_scatter(N, D, NPAD):
    info = plsc.get_sparse_core_info()
    NC, NS = info.num_cores, info.num_subcores
    NW = NC * NS
    n_per_w = N // NW
    CH = 64
    n_ch = n_per_w // CH

    @functools.partial(
        pl.kernel,
        out_type=[
            jax.ShapeDtypeStruct((NPAD, D), jnp.float32),
            jax.ShapeDtypeStruct((N,), jnp.int32),
        ],
        mesh=plsc.VectorSubcoreMesh(core_axis_name="c", subcore_axis_name="s"),
        compiler_params=pltpu.CompilerParams(needs_layout_passes=False),
        scratch_types=[
            pltpu.VMEM((16,), jnp.int32),
            pltpu.VMEM((CH,), jnp.int32), pltpu.VMEM((CH,), jnp.int32),
            pltpu.VMEM((CH,), jnp.int32), pltpu.VMEM((CH,), jnp.int32),
            pltpu.VMEM((CH, D), jnp.float32), pltpu.VMEM((CH, D), jnp.float32),
            pltpu.SemaphoreType.DMA, pltpu.SemaphoreType.DMA,
            pltpu.SemaphoreType.DMA, pltpu.SemaphoreType.DMA,
            pltpu.SemaphoreType.DMA, pltpu.SemaphoreType.DMA,
            pltpu.SemaphoreType.DMA, pltpu.SemaphoreType.DMA,
            pltpu.SemaphoreType.DMA,
        ],
    )
    def scatter_x(x_hbm, pack_hbm, gs_hbm, xs_hbm, dest_hbm,
                  gs_v, p_v0, p_v1, d_v0, d_v1, r_v0, r_v1,
                  sgs, sp0, sp1, sr0, sr1, ssc0, ssc1, sst0, sst1):
        wid = lax.axis_index("s") * NC + lax.axis_index("c")
        base = wid * n_per_w
        pltpu.sync_copy(gs_hbm, gs_v)
        bufs = [(p_v0, d_v0, r_v0, sp0, sr0, ssc0, sst0),
                (p_v1, d_v1, r_v1, sp1, sr1, ssc1, sst1)]
        pend_sc = [None, None]
        pend_st = [None, None]
        for ci in range(n_ch):
            b = ci % 2
            p_v, d_v, r_v, sp, sr, ssc, sst = bufs[b]
            if pend_sc[b] is not None:
                pend_sc[b].wait()
                pend_st[b].wait()
            off = base + ci * CH
            hp = pltpu.async_copy(pack_hbm.at[pl.ds(off, CH)], p_v, sp)
            hr = pltpu.async_copy(x_hbm.at[pl.ds(off, CH)], r_v, sr)
            hp.wait()
            for k in range(CH // 16):
                p16 = p_v[pl.ds(k * 16, 16)]
                e16 = lax.shift_right_logical(p16, RANK_BITS)
                g16 = lax.bitwise_and(p16, (1 << RANK_BITS) - 1)
                d_v[pl.ds(k * 16, 16)] = plsc.load_gather(gs_v, [e16]) + g16
            hr.wait()
            pend_sc[b] = pltpu.async_copy(r_v, xs_hbm.at[d_v], ssc)
            pend_st[b] = pltpu.async_copy(d_v, dest_hbm.at[pl.ds(off, CH)], sst)
        for h in pend_sc + pend_st:
            if h is not None:
                h.wait()

    return scatter_x


def _make_gather(N, C, NPAD):
    info = plsc.get_sparse_core_info()
    NC, NS = info.num_cores, info.num_subcores
    NW = NC * NS
    n_per_w = N // NW
    CH = 32
    n_ch = n_per_w // CH

    @functools.partial(
        pl.kernel,
        out_type=jax.ShapeDtypeStruct((N, C), jnp.float32),
        mesh=plsc.VectorSubcoreMesh(core_axis_name="c", subcore_axis_name="s"),
        compiler_params=pltpu.CompilerParams(needs_layout_passes=False),
        scratch_types=[
            pltpu.VMEM((CH,), jnp.int32), pltpu.VMEM((CH,), jnp.int32),
            pltpu.VMEM((CH,), jnp.int32),
            pltpu.VMEM((CH, C), jnp.float32), pltpu.VMEM((CH, C), jnp.float32),
            pltpu.VMEM((CH, C), jnp.float32),
            pltpu.SemaphoreType.DMA, pltpu.SemaphoreType.DMA,
            pltpu.SemaphoreType.DMA, pltpu.SemaphoreType.DMA,
            pltpu.SemaphoreType.DMA, pltpu.SemaphoreType.DMA,
            pltpu.SemaphoreType.DMA, pltpu.SemaphoreType.DMA,
            pltpu.SemaphoreType.DMA,
        ],
    )
    def gather_y(y_hbm, dest_hbm, out_hbm,
                 d_v0, d_v1, d_v2, r_v0, r_v1, r_v2,
                 sd0, sd1, sd2, sg0, sg1, sg2, st0, st1, st2):
        wid = lax.axis_index("s") * NC + lax.axis_index("c")
        base = wid * n_per_w
        bufs = [(d_v0, r_v0, sd0, sg0, st0),
                (d_v1, r_v1, sd1, sg1, st1),
                (d_v2, r_v2, sd2, sg2, st2)]
        pend_st = [None, None, None]
        for ci in range(n_ch):
            b = ci % 3
            d_v, r_v, sd, sg, st = bufs[b]
            if pend_st[b] is not None:
                pend_st[b].wait()
            off = base + ci * CH
            hd = pltpu.async_copy(dest_hbm.at[pl.ds(off, CH)], d_v, sd)
            hd.wait()
            hg = pltpu.async_copy(y_hbm.at[d_v], r_v, sg)
            hg.wait()
            pend_st[b] = pltpu.async_copy(r_v, out_hbm.at[pl.ds(off, CH)], st)
        for h in pend_st:
            if h is not None:
                h.wait()

    return gather_y


def _mm_body(be_ref, xs_ref, w_ref, b_ref, o_ref, ones_ref, wb_ref, *,
             nblk_pad):
    j = pl.program_id(0)
    used = be_ref[nblk_pad - 1]

    @pl.when(j == 0)
    def _():
        ones_ref[...] = jnp.ones_like(ones_ref)
        wb_ref[...] = w_ref[...].astype(jnp.bfloat16)

    @pl.when(j < used)
    def _():
        e = be_ref[j]
        xb = xs_ref[...].astype(jnp.bfloat16)
        y = jnp.dot(xb, wb_ref[e], preferred_element_type=jnp.float32)
        ey = jnp.exp(jnp.minimum(y + b_ref[e], 80.0))
        s = jnp.dot(ey, ones_ref[...], precision=lax.Precision.DEFAULT,
                    preferred_element_type=jnp.float32)
        o_ref[...] = ey * (1.0 / s[:, :1])


def _grouped_mm(be_arr, x_sorted, expert_W, expert_b, *, D, C, NPAD,
                nblk_pad):
    nblk = NPAD // BM
    E = expert_W.shape[0]
    last = nblk_pad - 1

    def _cap(j, be):
        return jnp.minimum(j, be[last] - 1)

    grid_spec = pltpu.PrefetchScalarGridSpec(
        num_scalar_prefetch=1,
        grid=(nblk,),
        in_specs=[
            pl.BlockSpec((BM, D), lambda j, be: (_cap(j, be), 0)),
            pl.BlockSpec((E, D, C), lambda j, be: (0, 0, 0)),
            pl.BlockSpec((E, 1, C), lambda j, be: (0, 0, 0)),
        ],
        out_specs=pl.BlockSpec((BM, C), lambda j, be: (_cap(j, be), 0)),
        scratch_shapes=[pltpu.VMEM((C, 128), jnp.float32),
                        pltpu.VMEM((E, D, C), jnp.bfloat16)],
    )
    return pl.pallas_call(
        functools.partial(_mm_body, nblk_pad=nblk_pad),
        grid_spec=grid_spec,
        out_shape=jax.ShapeDtypeStruct((NPAD, C), jnp.float32),
    )(be_arr, x_sorted, expert_W,
      expert_b.reshape(expert_b.shape[0], 1, C))


def kernel(inputs, expert_W, expert_b, gate_W, gate_b):
    N, D = inputs.shape
    E, _, C = expert_W.shape
    NPAD = N + E * BM
    nblk_pad = 128

    pack3, gs2, be2 = _route(
        inputs, gate_W, gate_b.reshape(1, E), N=N, D=D, E=E,
        nblk_pad=nblk_pad)
    pack = pack3.reshape(N)
    gs = gs2.reshape(16)
    be_arr = be2.reshape(nblk_pad)

    x_sorted, dest = _make_scatter(N, D, NPAD)(inputs, pack, gs)
    y_sorted = _grouped_mm(be_arr, x_sorted, expert_W, expert_b,
                           D=D, C=C, NPAD=NPAD, nblk_pad=nblk_pad)
    out = _make_gather(N, C, NPAD)(y_sorted, dest)
    return out


# speedup vs baseline: 1.2896x; 1.0709x over previous
import functools

import jax
import jax.numpy as jnp
from jax import lax
from jax.experimental import pallas as pl
from jax.experimental.pallas import tpu as pltpu
from jax.experimental.pallas import tpu_sc as plsc

BM = 512
RANK_BITS = 13


def _route_body(x_ref, gw_ref, gb_ref, pack_ref, x16_ref, gs_ref,
                be_ref, acc_ref, tri_ref, *, E, nblocks, nblk_pad):
    i = pl.program_id(0)

    @pl.when(i == 0)
    def _():
        acc_ref[...] = jnp.zeros_like(acc_ref)
        tr = lax.broadcasted_iota(jnp.int32, (BM, BM), 0)
        tc = lax.broadcasted_iota(jnp.int32, (BM, BM), 1)
        tri_ref[...] = (tc < tr).astype(jnp.float32)

    x = x_ref[...]
    ui = pltpu.bitcast(x.astype(jnp.bfloat16), jnp.uint16)
    dp = ui.shape[1] // 2
    lo = ui[:, :dp].astype(jnp.int32)
    hi = ui[:, dp:].astype(jnp.int32)
    x16_ref[...] = lo | (hi << 16)
    glog = jnp.dot(x, gw_ref[...], preferred_element_type=jnp.float32)
    glog = glog + gb_ref[...]
    m = jnp.max(glog, axis=-1, keepdims=True)
    hit = (glog == m).astype(jnp.float32)
    ue_r = lax.broadcasted_iota(jnp.int32, (E, E), 0)
    ue_c = lax.broadcasted_iota(jnp.int32, (E, E), 1)
    triu = (ue_r < ue_c).astype(jnp.float32)
    left = jnp.dot(hit, triu, preferred_element_type=jnp.float32)
    onehot = hit * (left == 0.0).astype(jnp.float32)
    ranks_local = jnp.dot(tri_ref[...], onehot,
                          preferred_element_type=jnp.float32)
    counts_prev = acc_ref[...]
    grank = jnp.sum(onehot * (ranks_local + counts_prev), axis=-1)
    acc_ref[...] = counts_prev + jnp.sum(onehot, axis=0, keepdims=True)
    e_iota = lax.broadcasted_iota(jnp.int32, (1, E), 1).astype(jnp.float32)
    eidx = jnp.sum(onehot * e_iota, axis=-1)
    packed = eidx.astype(jnp.int32) * (1 << RANK_BITS) + grank.astype(jnp.int32)
    pack_ref[...] = packed.reshape(1, 1, BM)

    @pl.when(i == nblocks - 1)
    def _finalize():
        counts = acc_ref[...]
        pad_blocks = jnp.ceil(counts / BM)
        tinc = (ue_r <= ue_c).astype(jnp.float32)
        ends = jnp.dot(pad_blocks, tinc,
                       preferred_element_type=jnp.float32)
        starts_rows = (ends - pad_blocks) * BM
        gs_ref[...] = jnp.concatenate(
            [starts_rows, jnp.zeros_like(starts_rows)],
            axis=-1).astype(jnp.int32)
        ends_i = ends.astype(jnp.int32)
        j_iota = lax.broadcasted_iota(jnp.int32, (nblk_pad, 1), 0)
        be = jnp.clip(jnp.sum((j_iota >= ends_i).astype(jnp.int32), axis=-1),
                      0, E - 1)
        used = ends_i[0, E - 1]
        jr = lax.broadcasted_iota(jnp.int32, (nblk_pad,), 0)
        be_ref[...] = jnp.where(jr == nblk_pad - 1, used,
                                be).reshape(1, nblk_pad)


def _route(inputs, gate_W, gate_b2, *, N, D, E, nblk_pad):
    nblocks = N // BM
    return pl.pallas_call(
        functools.partial(_route_body, E=E, nblocks=nblocks, nblk_pad=nblk_pad),
        grid=(nblocks,),
        in_specs=[
            pl.BlockSpec((BM, D), lambda i: (i, 0)),
            pl.BlockSpec((D, E), lambda i: (0, 0)),
            pl.BlockSpec((1, E), lambda i: (0, 0)),
        ],
        out_specs=[
            pl.BlockSpec((1, 1, BM), lambda i: (i, 0, 0)),
            pl.BlockSpec((BM, D // 2), lambda i: (i, 0)),
            pl.BlockSpec((1, 16), lambda i: (0, 0)),
            pl.BlockSpec((1, nblk_pad), lambda i: (0, 0)),
        ],
        out_shape=[
            jax.ShapeDtypeStruct((nblocks, 1, BM), jnp.int32),
            jax.ShapeDtypeStruct((N, D // 2), jnp.int32),
            jax.ShapeDtypeStruct((1, 16), jnp.int32),
            jax.ShapeDtypeStruct((1, nblk_pad), jnp.int32),
        ],
        scratch_shapes=[pltpu.VMEM((1, E), jnp.float32),
                        pltpu.VMEM((BM, BM), jnp.float32)],
    )(inputs, gate_W, gate_b2)


def _make_scatter(N, D, NPAD):
    DP = D // 2
    info = plsc.get_sparse_core_info()
    NC, NS = info.num_cores, info.num_subcores
    NW = NC * NS
    n_per_w = N // NW
    CH = 64
    n_ch = n_per_w // CH

    @functools.partial(
        pl.kernel,
        out_type=[
            jax.ShapeDtypeStruct((NPAD, DP), jnp.int32),
            jax.ShapeDtypeStruct((N,), jnp.int32),
        ],
        mesh=plsc.VectorSubcoreMesh(core_axis_name="c", subcore_axis_name="s"),
        compiler_params=pltpu.CompilerParams(needs_layout_passes=False),
        scratch_types=[
            pltpu.VMEM((16,), jnp.int32),
            pltpu.VMEM((CH,), jnp.int32), pltpu.VMEM((CH,), jnp.int32),
            pltpu.VMEM((CH,), jnp.int32), pltpu.VMEM((CH,), jnp.int32),
            pltpu.VMEM((CH, DP), jnp.int32), pltpu.VMEM((CH, DP), jnp.int32),
            pltpu.SemaphoreType.DMA, pltpu.SemaphoreType.DMA,
            pltpu.SemaphoreType.DMA, pltpu.SemaphoreType.DMA,
            pltpu.SemaphoreType.DMA, pltpu.SemaphoreType.DMA,
            pltpu.SemaphoreType.DMA, pltpu.SemaphoreType.DMA,
            pltpu.SemaphoreType.DMA,
        ],
    )
    def scatter_x(x_hbm, pack_hbm, gs_hbm, xs_hbm, dest_hbm,
                  gs_v, p_v0, p_v1, d_v0, d_v1, r_v0, r_v1,
                  sgs, sp0, sp1, sr0, sr1, ssc0, ssc1, sst0, sst1):
        wid = lax.axis_index("s") * NC + lax.axis_index("c")
        base = wid * n_per_w
        pltpu.sync_copy(gs_hbm, gs_v)
        bufs = [(p_v0, d_v0, r_v0, sp0, sr0, ssc0, sst0),
                (p_v1, d_v1, r_v1, sp1, sr1, ssc1, sst1)]
        pend_sc = [None, None]
        pend_st = [None, None]
        for ci in range(n_ch):
            b = ci % 2
            p_v, d_v, r_v, sp, sr, ssc, sst = bufs[b]
            if pend_sc[b] is not None:
                pend_sc[b].wait()
                pend_st[b].wait()
            off = base + ci * CH
            hp = pltpu.async_copy(pack_hbm.at[pl.ds(off, CH)], p_v, sp)
            hr = pltpu.async_copy(x_hbm.at[pl.ds(off, CH)], r_v, sr)
            hp.wait()
            for k in range(CH // 16):
                p16 = p_v[pl.ds(k * 16, 16)]
                e16 = lax.shift_right_logical(p16, RANK_BITS)
                g16 = lax.bitwise_and(p16, (1 << RANK_BITS) - 1)
                d_v[pl.ds(k * 16, 16)] = plsc.load_gather(gs_v, [e16]) + g16
            hr.wait()
            pend_sc[b] = pltpu.async_copy(r_v, xs_hbm.at[d_v], ssc)
            pend_st[b] = pltpu.async_copy(d_v, dest_hbm.at[pl.ds(off, CH)], sst)
        for h in pend_sc + pend_st:
            if h is not None:
                h.wait()

    return scatter_x


def _make_gather(N, C, NPAD):
    info = plsc.get_sparse_core_info()
    NC, NS = info.num_cores, info.num_subcores
    NW = NC * NS
    n_per_w = N // NW
    CH = 32
    n_ch = n_per_w // CH

    @functools.partial(
        pl.kernel,
        out_type=jax.ShapeDtypeStruct((N, C), jnp.float32),
        mesh=plsc.VectorSubcoreMesh(core_axis_name="c", subcore_axis_name="s"),
        compiler_params=pltpu.CompilerParams(needs_layout_passes=False),
        scratch_types=[
            pltpu.VMEM((CH,), jnp.int32), pltpu.VMEM((CH,), jnp.int32),
            pltpu.VMEM((CH,), jnp.int32),
            pltpu.VMEM((CH, C), jnp.float32), pltpu.VMEM((CH, C), jnp.float32),
            pltpu.VMEM((CH, C), jnp.float32),
            pltpu.SemaphoreType.DMA, pltpu.SemaphoreType.DMA,
            pltpu.SemaphoreType.DMA, pltpu.SemaphoreType.DMA,
            pltpu.SemaphoreType.DMA, pltpu.SemaphoreType.DMA,
            pltpu.SemaphoreType.DMA, pltpu.SemaphoreType.DMA,
            pltpu.SemaphoreType.DMA,
        ],
    )
    def gather_y(y_hbm, dest_hbm, out_hbm,
                 d_v0, d_v1, d_v2, r_v0, r_v1, r_v2,
                 sd0, sd1, sd2, sg0, sg1, sg2, st0, st1, st2):
        wid = lax.axis_index("s") * NC + lax.axis_index("c")
        base = wid * n_per_w
        bufs = [(d_v0, r_v0, sd0, sg0, st0),
                (d_v1, r_v1, sd1, sg1, st1),
                (d_v2, r_v2, sd2, sg2, st2)]
        pend_st = [None, None, None]
        for ci in range(n_ch):
            b = ci % 3
            d_v, r_v, sd, sg, st = bufs[b]
            if pend_st[b] is not None:
                pend_st[b].wait()
            off = base + ci * CH
            hd = pltpu.async_copy(dest_hbm.at[pl.ds(off, CH)], d_v, sd)
            hd.wait()
            hg = pltpu.async_copy(y_hbm.at[d_v], r_v, sg)
            hg.wait()
            pend_st[b] = pltpu.async_copy(r_v, out_hbm.at[pl.ds(off, CH)], st)
        for h in pend_st:
            if h is not None:
                h.wait()

    return gather_y


def _mm_body(be_ref, xs_ref, w_ref, b_ref, o_ref, ones_ref, wb_ref, *,
             nblk_pad):
    j = pl.program_id(0)
    used = be_ref[nblk_pad - 1]

    @pl.when(j == 0)
    def _():
        ones_ref[...] = jnp.ones_like(ones_ref)
        wb_ref[...] = w_ref[...].astype(jnp.bfloat16)

    @pl.when(j < used)
    def _():
        e = be_ref[j]
        xs = xs_ref[...]
        lo = (xs & 0xFFFF).astype(jnp.uint16)
        hi = lax.shift_right_logical(xs, 16).astype(jnp.uint16)
        xb = pltpu.bitcast(jnp.concatenate([lo, hi], axis=-1), jnp.bfloat16)
        y = jnp.dot(xb, wb_ref[e], preferred_element_type=jnp.float32)
        ey = jnp.exp(jnp.minimum(y + b_ref[e], 80.0))
        s = jnp.dot(ey, ones_ref[...], precision=lax.Precision.DEFAULT,
                    preferred_element_type=jnp.float32)
        o_ref[...] = ey * (1.0 / s[:, :1])


def _grouped_mm(be_arr, x_sorted, expert_W, expert_b, *, D, C, NPAD,
                nblk_pad):
    nblk = NPAD // BM
    E = expert_W.shape[0]
    last = nblk_pad - 1

    def _cap(j, be):
        return jnp.minimum(j, be[last] - 1)

    grid_spec = pltpu.PrefetchScalarGridSpec(
        num_scalar_prefetch=1,
        grid=(nblk,),
        in_specs=[
            pl.BlockSpec((BM, D // 2), lambda j, be: (_cap(j, be), 0)),
            pl.BlockSpec((E, D, C), lambda j, be: (0, 0, 0)),
            pl.BlockSpec((E, 1, C), lambda j, be: (0, 0, 0)),
        ],
        out_specs=pl.BlockSpec((BM, C), lambda j, be: (_cap(j, be), 0)),
        scratch_shapes=[pltpu.VMEM((C, 128), jnp.float32),
                        pltpu.VMEM((E, D, C), jnp.bfloat16)],
    )
    return pl.pallas_call(
        functools.partial(_mm_body, nblk_pad=nblk_pad),
        grid_spec=grid_spec,
        out_shape=jax.ShapeDtypeStruct((NPAD, C), jnp.float32),
    )(be_arr, x_sorted, expert_W,
      expert_b.reshape(expert_b.shape[0], 1, C))


def kernel(inputs, expert_W, expert_b, gate_W, gate_b):
    N, D = inputs.shape
    E, _, C = expert_W.shape
    NPAD = N + E * BM
    nblk_pad = 128

    pack3, x16, gs2, be2 = _route(
        inputs, gate_W, gate_b.reshape(1, E), N=N, D=D, E=E,
        nblk_pad=nblk_pad)
    pack = pack3.reshape(N)
    gs = gs2.reshape(16)
    be_arr = be2.reshape(nblk_pad)

    x_sorted, dest = _make_scatter(N, D, NPAD)(x16, pack, gs)
    y_sorted = _grouped_mm(be_arr, x_sorted, expert_W, expert_b,
                           D=D, C=C, NPAD=NPAD, nblk_pad=nblk_pad)
    out = _make_gather(N, C, NPAD)(y_sorted, dest)
    return out


# speedup vs baseline: 1.3320x; 1.0328x over previous
import functools

import jax
import jax.numpy as jnp
from jax import lax
from jax.experimental import pallas as pl
from jax.experimental.pallas import tpu as pltpu
from jax.experimental.pallas import tpu_sc as plsc

BM = 512
RANK_BITS = 13


def _route_body(x_ref, gw_ref, gb_ref, pack_ref, x16_ref, gs_ref,
                be_ref, acc_ref, tri_ref, *, E, nblocks, nblk_pad):
    i = pl.program_id(0)

    @pl.when(i == 0)
    def _():
        acc_ref[...] = jnp.zeros_like(acc_ref)
        tr = lax.broadcasted_iota(jnp.int32, (BM, BM), 0)
        tc = lax.broadcasted_iota(jnp.int32, (BM, BM), 1)
        tri_ref[...] = (tc < tr).astype(jnp.float32)

    x = x_ref[...]
    ui = pltpu.bitcast(x.astype(jnp.bfloat16), jnp.uint16)
    dp = ui.shape[1] // 2
    lo = ui[:, :dp].astype(jnp.int32)
    hi = ui[:, dp:].astype(jnp.int32)
    x16_ref[...] = lo | (hi << 16)
    glog = jnp.dot(x, gw_ref[...], preferred_element_type=jnp.float32)
    glog = glog + gb_ref[...]
    m = jnp.max(glog, axis=-1, keepdims=True)
    hit = (glog == m).astype(jnp.float32)
    ue_r = lax.broadcasted_iota(jnp.int32, (E, E), 0)
    ue_c = lax.broadcasted_iota(jnp.int32, (E, E), 1)
    triu = (ue_r < ue_c).astype(jnp.float32)
    left = jnp.dot(hit, triu, preferred_element_type=jnp.float32)
    onehot = hit * (left == 0.0).astype(jnp.float32)
    ranks_local = jnp.dot(tri_ref[...], onehot,
                          precision=lax.Precision.DEFAULT,
                          preferred_element_type=jnp.float32)
    counts_prev = acc_ref[...]
    grank = jnp.sum(onehot * (ranks_local + counts_prev), axis=-1)
    acc_ref[...] = counts_prev + jnp.sum(onehot, axis=0, keepdims=True)
    e_iota = lax.broadcasted_iota(jnp.int32, (1, E), 1).astype(jnp.float32)
    eidx = jnp.sum(onehot * e_iota, axis=-1)
    packed = eidx.astype(jnp.int32) * (1 << RANK_BITS) + grank.astype(jnp.int32)
    pack_ref[...] = packed.reshape(1, 1, BM)

    @pl.when(i == nblocks - 1)
    def _finalize():
        counts = acc_ref[...]
        pad_blocks = jnp.ceil(counts / BM)
        tinc = (ue_r <= ue_c).astype(jnp.float32)
        ends = jnp.dot(pad_blocks, tinc,
                       preferred_element_type=jnp.float32)
        starts_rows = (ends - pad_blocks) * BM
        gs_ref[...] = jnp.concatenate(
            [starts_rows, jnp.zeros_like(starts_rows)],
            axis=-1).astype(jnp.int32)
        ends_i = ends.astype(jnp.int32)
        j_iota = lax.broadcasted_iota(jnp.int32, (nblk_pad, 1), 0)
        be = jnp.clip(jnp.sum((j_iota >= ends_i).astype(jnp.int32), axis=-1),
                      0, E - 1)
        used = ends_i[0, E - 1]
        jr = lax.broadcasted_iota(jnp.int32, (nblk_pad,), 0)
        be_ref[...] = jnp.where(jr == nblk_pad - 1, used,
                                be).reshape(1, nblk_pad)


def _route(inputs, gate_W, gate_b2, *, N, D, E, nblk_pad):
    nblocks = N // BM
    return pl.pallas_call(
        functools.partial(_route_body, E=E, nblocks=nblocks, nblk_pad=nblk_pad),
        grid=(nblocks,),
        in_specs=[
            pl.BlockSpec((BM, D), lambda i: (i, 0)),
            pl.BlockSpec((D, E), lambda i: (0, 0)),
            pl.BlockSpec((1, E), lambda i: (0, 0)),
        ],
        out_specs=[
            pl.BlockSpec((1, 1, BM), lambda i: (i, 0, 0)),
            pl.BlockSpec((BM, D // 2), lambda i: (i, 0)),
            pl.BlockSpec((1, 16), lambda i: (0, 0)),
            pl.BlockSpec((1, nblk_pad), lambda i: (0, 0)),
        ],
        out_shape=[
            jax.ShapeDtypeStruct((nblocks, 1, BM), jnp.int32),
            jax.ShapeDtypeStruct((N, D // 2), jnp.int32),
            jax.ShapeDtypeStruct((1, 16), jnp.int32),
            jax.ShapeDtypeStruct((1, nblk_pad), jnp.int32),
        ],
        scratch_shapes=[pltpu.VMEM((1, E), jnp.float32),
                        pltpu.VMEM((BM, BM), jnp.float32)],
    )(inputs, gate_W, gate_b2)


def _make_scatter(N, D, NPAD):
    DP = D // 2
    info = plsc.get_sparse_core_info()
    NC, NS = info.num_cores, info.num_subcores
    NW = NC * NS
    n_per_w = N // NW
    CH = 64
    n_ch = n_per_w // CH

    @functools.partial(
        pl.kernel,
        out_type=[
            jax.ShapeDtypeStruct((NPAD, DP), jnp.int32),
            jax.ShapeDtypeStruct((N,), jnp.int32),
        ],
        mesh=plsc.VectorSubcoreMesh(core_axis_name="c", subcore_axis_name="s"),
        compiler_params=pltpu.CompilerParams(needs_layout_passes=False),
        scratch_types=[
            pltpu.VMEM((16,), jnp.int32),
            pltpu.VMEM((CH,), jnp.int32), pltpu.VMEM((CH,), jnp.int32),
            pltpu.VMEM((CH,), jnp.int32), pltpu.VMEM((CH,), jnp.int32),
            pltpu.VMEM((CH, DP), jnp.int32), pltpu.VMEM((CH, DP), jnp.int32),
            pltpu.SemaphoreType.DMA, pltpu.SemaphoreType.DMA,
            pltpu.SemaphoreType.DMA, pltpu.SemaphoreType.DMA,
            pltpu.SemaphoreType.DMA, pltpu.SemaphoreType.DMA,
            pltpu.SemaphoreType.DMA, pltpu.SemaphoreType.DMA,
            pltpu.SemaphoreType.DMA,
        ],
    )
    def scatter_x(x_hbm, pack_hbm, gs_hbm, xs_hbm, dest_hbm,
                  gs_v, p_v0, p_v1, d_v0, d_v1, r_v0, r_v1,
                  sgs, sp0, sp1, sr0, sr1, ssc0, ssc1, sst0, sst1):
        wid = lax.axis_index("s") * NC + lax.axis_index("c")
        base = wid * n_per_w
        pltpu.sync_copy(gs_hbm, gs_v)
        bufs = [(p_v0, d_v0, r_v0, sp0, sr0, ssc0, sst0),
                (p_v1, d_v1, r_v1, sp1, sr1, ssc1, sst1)]
        pend_sc = [None, None]
        pend_st = [None, None]
        for ci in range(n_ch):
            b = ci % 2
            p_v, d_v, r_v, sp, sr, ssc, sst = bufs[b]
            if pend_sc[b] is not None:
                pend_sc[b].wait()
                pend_st[b].wait()
            off = base + ci * CH
            hp = pltpu.async_copy(pack_hbm.at[pl.ds(off, CH)], p_v, sp)
            hr = pltpu.async_copy(x_hbm.at[pl.ds(off, CH)], r_v, sr)
            hp.wait()
            for k in range(CH // 16):
                p16 = p_v[pl.ds(k * 16, 16)]
                e16 = lax.shift_right_logical(p16, RANK_BITS)
                g16 = lax.bitwise_and(p16, (1 << RANK_BITS) - 1)
                d_v[pl.ds(k * 16, 16)] = plsc.load_gather(gs_v, [e16]) + g16
            hr.wait()
            pend_sc[b] = pltpu.async_copy(r_v, xs_hbm.at[d_v], ssc)
            pend_st[b] = pltpu.async_copy(d_v, dest_hbm.at[pl.ds(off, CH)], sst)
        for h in pend_sc + pend_st:
            if h is not None:
                h.wait()

    return scatter_x


def _make_gather(N, C, NPAD):
    info = plsc.get_sparse_core_info()
    NC, NS = info.num_cores, info.num_subcores
    NW = NC * NS
    n_per_w = N // NW
    CH = 32
    n_ch = n_per_w // CH

    @functools.partial(
        pl.kernel,
        out_type=jax.ShapeDtypeStruct((N, C), jnp.float32),
        mesh=plsc.VectorSubcoreMesh(core_axis_name="c", subcore_axis_name="s"),
        compiler_params=pltpu.CompilerParams(needs_layout_passes=False),
        scratch_types=[
            pltpu.VMEM((CH,), jnp.int32), pltpu.VMEM((CH,), jnp.int32),
            pltpu.VMEM((CH,), jnp.int32),
            pltpu.VMEM((CH, C), jnp.float32), pltpu.VMEM((CH, C), jnp.float32),
            pltpu.VMEM((CH, C), jnp.float32),
            pltpu.SemaphoreType.DMA, pltpu.SemaphoreType.DMA,
            pltpu.SemaphoreType.DMA, pltpu.SemaphoreType.DMA,
            pltpu.SemaphoreType.DMA, pltpu.SemaphoreType.DMA,
            pltpu.SemaphoreType.DMA, pltpu.SemaphoreType.DMA,
            pltpu.SemaphoreType.DMA,
        ],
    )
    def gather_y(y_hbm, dest_hbm, out_hbm,
                 d_v0, d_v1, d_v2, r_v0, r_v1, r_v2,
                 sd0, sd1, sd2, sg0, sg1, sg2, st0, st1, st2):
        wid = lax.axis_index("s") * NC + lax.axis_index("c")
        base = wid * n_per_w
        bufs = [(d_v0, r_v0, sd0, sg0, st0),
                (d_v1, r_v1, sd1, sg1, st1),
                (d_v2, r_v2, sd2, sg2, st2)]
        pend_st = [None, None, None]
        pend_g = [None, None, None]
        for ci in range(n_ch):
            b = ci % 3
            d_v, r_v, sd, sg, st = bufs[b]
            if pend_st[b] is not None:
                pend_st[b].wait()
            off = base + ci * CH
            hd = pltpu.async_copy(dest_hbm.at[pl.ds(off, CH)], d_v, sd)
            hd.wait()
            pend_g[b] = pltpu.async_copy(y_hbm.at[d_v], r_v, sg)
            if ci >= 1:
                bp = (ci - 1) % 3
                pend_g[bp].wait()
                pend_st[bp] = pltpu.async_copy(
                    bufs[bp][1], out_hbm.at[pl.ds(base + (ci - 1) * CH, CH)],
                    bufs[bp][4])
        bl = (n_ch - 1) % 3
        pend_g[bl].wait()
        pend_st[bl] = pltpu.async_copy(
            bufs[bl][1], out_hbm.at[pl.ds(base + (n_ch - 1) * CH, CH)],
            bufs[bl][4])
        for h in pend_st:
            if h is not None:
                h.wait()

    return gather_y


def _mm_body(be_ref, xs_ref, w_ref, b_ref, o_ref, ones_ref, wb_ref, *,
             nblk_pad):
    j = pl.program_id(0)
    used = be_ref[nblk_pad - 1]

    @pl.when(j == 0)
    def _():
        ones_ref[...] = jnp.ones_like(ones_ref)
        wb_ref[...] = w_ref[...].astype(jnp.bfloat16)

    @pl.when(j < used)
    def _():
        e = be_ref[j]
        xs = xs_ref[...]
        lo = (xs & 0xFFFF).astype(jnp.uint16)
        hi = lax.shift_right_logical(xs, 16).astype(jnp.uint16)
        xb = pltpu.bitcast(jnp.concatenate([lo, hi], axis=-1), jnp.bfloat16)
        y = jnp.dot(xb, wb_ref[e], preferred_element_type=jnp.float32)
        ey = jnp.exp(jnp.minimum(y + b_ref[e], 80.0))
        s = jnp.dot(ey, ones_ref[...], precision=lax.Precision.DEFAULT,
                    preferred_element_type=jnp.float32)
        o_ref[...] = ey * (1.0 / s[:, :1])


def _grouped_mm(be_arr, x_sorted, expert_W, expert_b, *, D, C, NPAD,
                nblk_pad):
    nblk = NPAD // BM
    E = expert_W.shape[0]
    last = nblk_pad - 1

    def _cap(j, be):
        return jnp.minimum(j, be[last] - 1)

    grid_spec = pltpu.PrefetchScalarGridSpec(
        num_scalar_prefetch=1,
        grid=(nblk,),
        in_specs=[
            pl.BlockSpec((BM, D // 2), lambda j, be: (_cap(j, be), 0)),
            pl.BlockSpec((E, D, C), lambda j, be: (0, 0, 0)),
            pl.BlockSpec((E, 1, C), lambda j, be: (0, 0, 0)),
        ],
        out_specs=pl.BlockSpec((BM, C), lambda j, be: (_cap(j, be), 0)),
        scratch_shapes=[pltpu.VMEM((C, 128), jnp.float32),
                        pltpu.VMEM((E, D, C), jnp.bfloat16)],
    )
    return pl.pallas_call(
        functools.partial(_mm_body, nblk_pad=nblk_pad),
        grid_spec=grid_spec,
        out_shape=jax.ShapeDtypeStruct((NPAD, C), jnp.float32),
    )(be_arr, x_sorted, expert_W,
      expert_b.reshape(expert_b.shape[0], 1, C))


def kernel(inputs, expert_W, expert_b, gate_W, gate_b):
    N, D = inputs.shape
    E, _, C = expert_W.shape
    NPAD = N + E * BM
    nblk_pad = 128

    pack3, x16, gs2, be2 = _route(
        inputs, gate_W, gate_b.reshape(1, E), N=N, D=D, E=E,
        nblk_pad=nblk_pad)
    pack = pack3.reshape(N)
    gs = gs2.reshape(16)
    be_arr = be2.reshape(nblk_pad)

    x_sorted, dest = _make_scatter(N, D, NPAD)(x16, pack, gs)
    y_sorted = _grouped_mm(be_arr, x_sorted, expert_W, expert_b,
                           D=D, C=C, NPAD=NPAD, nblk_pad=nblk_pad)
    out = _make_gather(N, C, NPAD)(y_sorted, dest)
    return out
